# baseline (speedup 1.0000x reference)
import jax
import jax.numpy as jnp
from jax.experimental import pallas as pl
from jax.experimental.pallas import tpu as pltpu


def _fused_mlp_kernel(x_ref, gw_ref, uw_ref, dw_ref, rw_ref,
                      gb_ref, ub_ref, db_ref, rb_ref, o_ref):
    j = pl.program_id(1)
    nj = pl.num_programs(1)

    x = x_ref[...]
    gate = jnp.dot(x, gw_ref[...].T, preferred_element_type=jnp.float32)
    gate = gate + gb_ref[...]
    up = jnp.dot(x, uw_ref[...].T, preferred_element_type=jnp.float32)
    up = up + ub_ref[...]
    h = gate * (up * jax.nn.sigmoid(up))
    part = jax.lax.dot_general(h, dw_ref[...], (((1,), (1,)), ((), ())),
                               preferred_element_type=jnp.float32)

    @pl.when(j == 0)
    def _():
        o_ref[...] = part

    @pl.when(j > 0)
    def _():
        o_ref[...] = o_ref[...] + part

    @pl.when(j == nj - 1)
    def _():
        logits = jnp.dot(x, rw_ref[...].T,
                         preferred_element_type=jnp.float32) + rb_ref[...]
        m1 = jnp.max(logits, axis=1, keepdims=True)
        masked = jnp.where(logits >= m1, -jnp.inf, logits)
        m2 = jnp.max(masked, axis=1, keepdims=True)
        e2 = jnp.exp(m2 - m1)
        denom = 1.0 + e2
        w = 1.0 / denom + e2 / denom
        o_ref[...] = (o_ref[...] + db_ref[...]) * w


def kernel(hidden_states, router_w, router_b, gate_up_w, gate_up_b,
           down_w, down_b):
    T, H = hidden_states.shape
    E = router_w.shape[0]
    I = down_w.shape[1]

    BT = 512
    BJ = 512
    nt = T // BT
    nj = I // BJ

    gate_up_b2 = gate_up_b.reshape(1, 2 * I)
    down_b2 = down_b.reshape(1, H)
    router_b2 = router_b.reshape(1, E)

    grid = (nt, nj)
    out = pl.pallas_call(
        _fused_mlp_kernel,
        grid=grid,
        in_specs=[
            pl.BlockSpec((BT, H), lambda t, j: (t, 0)),
            pl.BlockSpec((BJ, H), lambda t, j: (j, 0)),
            pl.BlockSpec((BJ, H), lambda t, j, _nj=nj: (_nj + j, 0)),
            pl.BlockSpec((H, BJ), lambda t, j: (0, j)),
            pl.BlockSpec((E, H), lambda t, j: (0, 0)),
            pl.BlockSpec((1, BJ), lambda t, j: (0, j)),
            pl.BlockSpec((1, BJ), lambda t, j, _nj=nj: (0, _nj + j)),
            pl.BlockSpec((1, H), lambda t, j: (0, 0)),
            pl.BlockSpec((1, E), lambda t, j: (0, 0)),
        ],
        out_specs=pl.BlockSpec((BT, H), lambda t, j: (t, 0)),
        out_shape=jax.ShapeDtypeStruct((T, H), jnp.float32),
        compiler_params=pltpu.CompilerParams(
            dimension_semantics=("parallel", "arbitrary"),
        ),
    )(hidden_states, gate_up_w, gate_up_w, down_w, router_w,
      gate_up_b2, gate_up_b2, down_b2, router_b2)
    return out
